# Initial kernel scaffold; baseline (speedup 1.0000x reference)
import jax
import jax.numpy as jnp
from jax.experimental import pallas as pl


def kernel(x, edge_index, W1, b1, g1, be1, W2, b2, g2, be2, W3, b3, g3, be3, W4, b4, g4, be4, W5, b5):
    raise NotImplementedError("write your pallas kernel here")


# speedup vs baseline: 4.2172x; 4.2172x over previous
import functools

import jax
import jax.numpy as jnp
from jax import lax
from jax.experimental import pallas as pl
from jax.experimental.pallas import tpu as pltpu
from jax.experimental.pallas import tpu_sc as plsc

N = 10000
E = 320000
NP = 10240
EP = 327680
ER = EP // 128
NSUB = 16
NCORE = 2
RPT_ALL = ER // NSUB
RPT_HALF = ER // (NCORE * NSUB)
BN_EPS = 1e-5
ZROWS = NP // NSUB
BT = 512
GT = NP // BT
W = 128


@functools.cache
def _sc_kernels():
    mesh = plsc.VectorSubcoreMesh(
        core_axis_name="c", subcore_axis_name="s", num_cores=NCORE, num_subcores=NSUB
    )

    out_t = jax.ShapeDtypeStruct((NCORE, NP, W), jnp.float32)
    scratch = [
        pltpu.VMEM((8, 128), jnp.int32),
        pltpu.VMEM((8, 128), jnp.int32),
        pltpu.VMEM((256, W), jnp.float32),
        pltpu.VMEM_SHARED((NP, W), jnp.float32),
        pltpu.SemaphoreType.DMA,
        pltpu.SemaphoreType.DMA,
    ]

    def zero_acc(zeros_hbm, acc, s):
        for k in range(ZROWS // 128):
            r = pl.multiple_of(s * ZROWS + k * 128, 128)
            pltpu.sync_copy(zeros_hbm, acc.at[pl.ds(r, 128)])

    def writeout(acc, out_hbm, c, s):
        r = pl.multiple_of(s * ZROWS, 128)
        pltpu.sync_copy(acc.at[pl.ds(r, ZROWS)], out_hbm.at[c, pl.ds(r, ZROWS)])

    def group(table_hbm, srcs2d, dsts2d, sidx, didx, buf, acc, sems):
        pltpu.sync_copy(srcs2d, sidx)
        pltpu.sync_copy(dsts2d, didx)
        cps = [None, None]
        for j in range(8):
            slot = j % 2
            cps[slot] = pltpu.async_copy(
                table_hbm.at[sidx.at[j]], buf.at[pl.ds(slot * 128, 128)], sems[slot]
            )
            if j > 0:
                prev = (j - 1) % 2
                cps[prev].wait()
                pltpu.sync_copy(
                    buf.at[pl.ds(prev * 128, 128)], acc.at[didx.at[j - 1]], add=True
                )
        cps[1].wait()
        pltpu.sync_copy(buf.at[pl.ds(128, 128)], acc.at[didx.at[7]], add=True)

    @functools.partial(pl.kernel, out_type=out_t, mesh=mesh, scratch_types=scratch)
    def scat_cols(table_hbm, srcs_hbm, dsts_hbm, zeros_hbm, out_hbm,
                  sidx, didx, buf, acc, sem0, sem1):
        c = lax.axis_index("c")
        s = lax.axis_index("s")
        zero_acc(zeros_hbm, acc, s)
        plsc.subcore_barrier()

        def gbody(g, carry):
            r0 = pl.multiple_of(s * RPT_ALL + g * 8, 8)
            group(table_hbm, srcs_hbm.at[c, pl.ds(r0, 8)], dsts_hbm.at[pl.ds(r0, 8)],
                  sidx, didx, buf, acc, (sem0, sem1))
            return carry

        lax.fori_loop(0, RPT_ALL // 8, gbody, 0)
        plsc.subcore_barrier()
        writeout(acc, out_hbm, c, s)

    @functools.partial(pl.kernel, out_type=out_t, mesh=mesh, scratch_types=scratch)
    def scat_edges(table_hbm, srcs_hbm, dsts_hbm, zeros_hbm, out_hbm,
                   sidx, didx, buf, acc, sem0, sem1):
        c = lax.axis_index("c")
        s = lax.axis_index("s")
        zero_acc(zeros_hbm, acc, s)
        plsc.subcore_barrier()
        base = (c * NSUB + s) * RPT_HALF

        def gbody(g, carry):
            r0 = pl.multiple_of(base + g * 8, 8)
            group(table_hbm, srcs_hbm.at[pl.ds(r0, 8)], dsts_hbm.at[pl.ds(r0, 8)],
                  sidx, didx, buf, acc, (sem0, sem1))
            return carry

        lax.fori_loop(0, RPT_HALF // 8, gbody, 0)
        plsc.subcore_barrier()
        writeout(acc, out_hbm, c, s)

    return scat_cols, scat_edges


def _prep_body(deg2_ref, x_ref, dinv_ref, t1_ref):
    deg = deg2_ref[0, :, 0:16] + deg2_ref[1, :, 0:16]
    dinv = lax.rsqrt(deg + 1.0)
    dinv_ref[...] = dinv
    t1_ref[...] = x_ref[...] * dinv[:, 0:1]


def _layer1_body(agg_ref, t_ref, dinv_ref, w_ref, b_ref, out_ref):
    dinv = dinv_ref[:, 0:1]
    u = (agg_ref[0] + agg_ref[1] + t_ref[...]) * dinv
    v = jnp.maximum(jnp.dot(u, w_ref[...], preferred_element_type=jnp.float32) + b_ref[...], 0.0)
    tn = v * dinv
    out_ref[0] = tn[:, :128]
    out_ref[1] = tn[:, 128:]


def _layer_body(agg_ref, t_ref, dinv_ref, w_ref, b_ref, out_ref):
    dinv = dinv_ref[:, 0:1]
    u = jnp.concatenate([agg_ref[0] + t_ref[0], agg_ref[1] + t_ref[1]], axis=1) * dinv
    v = jnp.maximum(jnp.dot(u, w_ref[...], preferred_element_type=jnp.float32) + b_ref[...], 0.0)
    tn = v * dinv
    out_ref[0] = tn[:, :128]
    out_ref[1] = tn[:, 128:]


def _layer4_body(agg_ref, t_ref, dinv_ref, w_ref, b_ref, w5_ref, out_ref):
    dinv = dinv_ref[:, 0:1]
    u = jnp.concatenate([agg_ref[0] + t_ref[0], agg_ref[1] + t_ref[1]], axis=1) * dinv
    v = jnp.maximum(jnp.dot(u, w_ref[...], preferred_element_type=jnp.float32) + b_ref[...], 0.0)
    m = jnp.dot(v, w5_ref[...], preferred_element_type=jnp.float32)
    tn = m * dinv
    out_ref[...] = jnp.concatenate([tn, jnp.zeros_like(tn)], axis=1)


def _final_body(agg_ref, t_ref, dinv_ref, b_ref, out_ref):
    dinv = dinv_ref[:, 0:1]
    u = ((agg_ref[0] + agg_ref[1] + t_ref[...]) * dinv)[:, :64] + b_ref[...]
    u = u - jnp.max(u, axis=1, keepdims=True)
    out_ref[...] = u - jnp.log(jnp.sum(jnp.exp(u), axis=1, keepdims=True))


def _row_spec(w):
    return pl.BlockSpec((BT, w), lambda i: (i, 0))


def _half_spec(w):
    return pl.BlockSpec((NCORE, BT, w), lambda i: (0, i, 0))


def _full_spec(shape):
    return pl.BlockSpec(shape, lambda i: tuple(0 for _ in shape))


def _tc_prep(deg2, xp):
    return pl.pallas_call(
        _prep_body,
        grid=(GT,),
        in_specs=[_half_spec(W), _row_spec(128)],
        out_specs=[_row_spec(16), _row_spec(128)],
        out_shape=[
            jax.ShapeDtypeStruct((NP, 16), jnp.float32),
            jax.ShapeDtypeStruct((NP, 128), jnp.float32),
        ],
    )(deg2, xp)


def _tc_layer1(agg, t, dinv, w, b):
    return pl.pallas_call(
        _layer1_body,
        grid=(GT,),
        in_specs=[
            _half_spec(W),
            _row_spec(128),
            _row_spec(16),
            _full_spec((128, 256)),
            _full_spec((1, 256)),
        ],
        out_specs=_half_spec(W),
        out_shape=jax.ShapeDtypeStruct((NCORE, NP, W), jnp.float32),
    )(agg, t, dinv, w, b)


def _tc_layer(agg, t, dinv, w, b):
    return pl.pallas_call(
        _layer_body,
        grid=(GT,),
        in_specs=[
            _half_spec(W),
            _half_spec(W),
            _row_spec(16),
            _full_spec((256, 256)),
            _full_spec((1, 256)),
        ],
        out_specs=_half_spec(W),
        out_shape=jax.ShapeDtypeStruct((NCORE, NP, W), jnp.float32),
    )(agg, t, dinv, w, b)


def _tc_layer4(agg, t, dinv, w, b, w5):
    return pl.pallas_call(
        _layer4_body,
        grid=(GT,),
        in_specs=[
            _half_spec(W),
            _half_spec(W),
            _row_spec(16),
            _full_spec((256, 256)),
            _full_spec((1, 256)),
            _full_spec((256, 64)),
        ],
        out_specs=_row_spec(128),
        out_shape=jax.ShapeDtypeStruct((NP, 128), jnp.float32),
    )(agg, t, dinv, w, b, w5)


def _tc_final(agg, t, dinv, b):
    return pl.pallas_call(
        _final_body,
        grid=(GT,),
        in_specs=[_half_spec(W), _row_spec(128), _row_spec(16), _full_spec((1, 64))],
        out_specs=_row_spec(64),
        out_shape=jax.ShapeDtypeStruct((NP, 64), jnp.float32),
    )(agg, t, dinv, b)


def kernel(x, edge_index, W1, b1, g1, be1, W2, b2, g2, be2, W3, b3, g3, be3,
           W4, b4, g4, be4, W5, b5):
    src = edge_index[0].astype(jnp.int32)
    dst = edge_index[1].astype(jnp.int32)
    padi = jnp.full((EP - E,), NP - 1, dtype=jnp.int32)
    srcp = jnp.concatenate([src, padi])
    dstp = jnp.concatenate([dst, padi])
    srcs1 = srcp.reshape(ER, 128)
    srcs2 = jnp.stack([srcp, srcp + NP]).reshape(NCORE, ER, 128)
    dsts = dstp.reshape(ER, 128)

    xp = jnp.pad(x, ((0, NP - N), (0, 0)))
    ones = jnp.ones((NP, W), jnp.float32)
    zeros = jnp.zeros((128, W), jnp.float32)

    def fold(wi, bi, gi, bei):
        sc = gi / jnp.sqrt(1.0 + BN_EPS)
        return wi * sc[None, :], (bi * sc + bei)[None, :]

    W1f, b1f = fold(W1, b1, g1, be1)
    W2f, b2f = fold(W2, b2, g2, be2)
    W3f, b3f = fold(W3, b3, g3, be3)
    W4f, b4f = fold(W4, b4, g4, be4)

    scat_cols, scat_edges = _sc_kernels()

    deg2 = scat_edges(ones, srcs1, dsts, zeros)
    dinv, t1 = _tc_prep(deg2, xp)

    agg1 = scat_edges(t1, srcs1, dsts, zeros)
    t2 = _tc_layer1(agg1, t1, dinv, W1f, b1f)

    agg2 = scat_cols(t2.reshape(NCORE * NP, W), srcs2, dsts, zeros)
    t3 = _tc_layer(agg2, t2, dinv, W2f, b2f)

    agg3 = scat_cols(t3.reshape(NCORE * NP, W), srcs2, dsts, zeros)
    t4 = _tc_layer(agg3, t3, dinv, W3f, b3f)

    agg4 = scat_cols(t4.reshape(NCORE * NP, W), srcs2, dsts, zeros)
    t5 = _tc_layer4(agg4, t4, dinv, W4f, b4f, W5)

    agg5 = scat_edges(t5, srcs1, dsts, zeros)
    out = _tc_final(agg5, t5, dinv, b5.reshape(1, 64))
    return out[:N]


# speedup vs baseline: 5.0298x; 1.1927x over previous
import functools

import jax
import jax.numpy as jnp
from jax import lax
from jax.experimental import pallas as pl
from jax.experimental.pallas import tpu as pltpu
from jax.experimental.pallas import tpu_sc as plsc

N = 10000
E = 320000
NP = 10240
EP = 327680
ER = EP // 128
NSUB = 16
NCORE = 2
RPT_ALL = ER // NSUB
RPT_HALF = ER // (NCORE * NSUB)
BN_EPS = 1e-5
ZROWS = NP // NSUB
BT = 512
GT = NP // BT
W = 128


NSLOT = 2


@functools.cache
def _sc_kernels():
    mesh = plsc.VectorSubcoreMesh(
        core_axis_name="c", subcore_axis_name="s", num_cores=NCORE, num_subcores=NSUB
    )

    out_t = jax.ShapeDtypeStruct((NCORE, NP, W), jnp.float32)

    def zero_acc(zeros_hbm, buf, acc, s):
        del buf
        for k in range(ZROWS // 128):
            r = pl.multiple_of(s * ZROWS + k * 128, 128)
            pltpu.sync_copy(zeros_hbm, acc.at[pl.ds(r, 128)])

    def writeout(acc, out_hbm, c, s):
        r = pl.multiple_of(s * ZROWS, 128)
        pltpu.sync_copy(acc.at[pl.ds(r, ZROWS)], out_hbm.at[c, pl.ds(r, ZROWS)])

    def make_scatter(rows_per_tile, col_split):
        ngrp = rows_per_tile // 8

        @functools.partial(
            pl.kernel,
            out_type=out_t,
            mesh=mesh,
            scratch_types=[
                pltpu.VMEM((8, 128), jnp.int32),
                pltpu.VMEM((8, 128), jnp.int32),
                pltpu.VMEM((NSLOT * 128, W), jnp.float32),
                pltpu.VMEM_SHARED((NP, W), jnp.float32),
            ]
            + [pltpu.SemaphoreType.DMA] * (2 * NSLOT),
        )
        def scat(table_hbm, srcs_hbm, dsts_hbm, zeros_hbm, out_hbm,
                 sidx, didx, buf, acc, *sems):
            gsem = sems[:NSLOT]
            ssem = sems[NSLOT:]
            c = lax.axis_index("c")
            s = lax.axis_index("s")
            zero_acc(zeros_hbm, buf, acc, s)
            plsc.subcore_barrier()

            def it(g, carry):
                if col_split:
                    w0 = pl.multiple_of(s * rows_per_tile + g * 8, 8)
                    pltpu.sync_copy(srcs_hbm.at[c, pl.ds(w0, 8)], sidx)
                else:
                    w0 = pl.multiple_of(
                        (c * NSUB + s) * rows_per_tile + g * 8, 8)
                    pltpu.sync_copy(srcs_hbm.at[pl.ds(w0, 8)], sidx)
                pltpu.sync_copy(dsts_hbm.at[pl.ds(w0, 8)], didx)
                cps = [None, None]
                for j in range(8):
                    k = j % 2
                    if j >= 2:
                        pltpu.make_async_copy(
                            zeros_hbm, buf.at[pl.ds(k * 128, 128)], ssem[k]
                        ).wait()
                    else:
                        @pl.when(g > 0)
                        def _drain(k=k):
                            pltpu.make_async_copy(
                                zeros_hbm, buf.at[pl.ds(k * 128, 128)], ssem[k]
                            ).wait()

                    cps[k] = pltpu.async_copy(
                        table_hbm.at[sidx.at[j]], buf.at[pl.ds(k * 128, 128)],
                        gsem[k],
                    )
                    if j >= 1:
                        p = (j - 1) % 2
                        cps[p].wait()
                        pltpu.async_copy(
                            buf.at[pl.ds(p * 128, 128)], acc.at[didx.at[j - 1]],
                            ssem[p], add=True,
                        )
                cps[1].wait()
                pltpu.async_copy(
                    buf.at[pl.ds(128, 128)], acc.at[didx.at[7]], ssem[1], add=True,
                )
                return carry

            lax.fori_loop(0, ngrp, it, 0)
            for k in range(NSLOT):
                pltpu.make_async_copy(
                    zeros_hbm, buf.at[pl.ds(k * 128, 128)], ssem[k]
                ).wait()
            plsc.subcore_barrier()
            writeout(acc, out_hbm, c, s)

        return scat

    @functools.partial(
        pl.kernel,
        out_type=out_t,
        mesh=mesh,
        scratch_types=[
            pltpu.VMEM((8, 128), jnp.int32),
            pltpu.VMEM((128, W), jnp.float32),
            pltpu.VMEM((128, W), jnp.float32),
            pltpu.VMEM_SHARED((NP, W), jnp.float32),
        ]
        + [pltpu.SemaphoreType.DMA] * NSLOT,
    )
    def deg_kernel(dsts_hbm, ones_hbm, zeros_hbm, out_hbm, didx, onesb,
                   zbuf, acc, *ssem):
        c = lax.axis_index("c")
        s = lax.axis_index("s")
        zero_acc(zeros_hbm, zbuf, acc, s)
        pltpu.sync_copy(ones_hbm, onesb)
        base = pl.multiple_of((c * NSUB + s) * RPT_HALF, 8)
        plsc.subcore_barrier()

        def it(g, carry):
            w0 = pl.multiple_of(base + g * 8, 8)
            pltpu.sync_copy(dsts_hbm.at[pl.ds(w0, 8)], didx)
            for j in range(8):
                k = j % NSLOT
                if j < NSLOT:
                    @pl.when(g > 0)
                    def _drain(k=k):
                        pltpu.make_async_copy(zeros_hbm, onesb, ssem[k]).wait()
                else:
                    pltpu.make_async_copy(zeros_hbm, onesb, ssem[k]).wait()
                pltpu.async_copy(onesb, acc.at[didx.at[j]], ssem[k], add=True)
            return carry

        lax.fori_loop(0, RPT_HALF // 8, it, 0)
        for k in range(NSLOT):
            pltpu.make_async_copy(zeros_hbm, onesb, ssem[k]).wait()
        plsc.subcore_barrier()
        writeout(acc, out_hbm, c, s)

    return make_scatter(RPT_ALL, True), make_scatter(RPT_HALF, False), deg_kernel


def _prep_body(deg2_ref, x_ref, dinv_ref, t1_ref):
    deg = deg2_ref[0, :, 0:16] + deg2_ref[1, :, 0:16]
    dinv = lax.rsqrt(deg + 1.0)
    dinv_ref[...] = dinv
    t1_ref[...] = x_ref[...] * dinv[:, 0:1]


def _layer1_body(agg_ref, t_ref, dinv_ref, w_ref, b_ref, out_ref):
    dinv = dinv_ref[:, 0:1]
    u = (agg_ref[0] + agg_ref[1] + t_ref[...]) * dinv
    v = jnp.maximum(jnp.dot(u, w_ref[...], preferred_element_type=jnp.float32) + b_ref[...], 0.0)
    tn = v * dinv
    out_ref[0] = tn[:, :128]
    out_ref[1] = tn[:, 128:]


def _layer_body(agg_ref, t_ref, dinv_ref, w_ref, b_ref, out_ref):
    dinv = dinv_ref[:, 0:1]
    u = jnp.concatenate([agg_ref[0] + t_ref[0], agg_ref[1] + t_ref[1]], axis=1) * dinv
    v = jnp.maximum(jnp.dot(u, w_ref[...], preferred_element_type=jnp.float32) + b_ref[...], 0.0)
    tn = v * dinv
    out_ref[0] = tn[:, :128]
    out_ref[1] = tn[:, 128:]


def _layer4_body(agg_ref, t_ref, dinv_ref, w_ref, b_ref, w5_ref, out_ref):
    dinv = dinv_ref[:, 0:1]
    u = jnp.concatenate([agg_ref[0] + t_ref[0], agg_ref[1] + t_ref[1]], axis=1) * dinv
    v = jnp.maximum(jnp.dot(u, w_ref[...], preferred_element_type=jnp.float32) + b_ref[...], 0.0)
    m = jnp.dot(v, w5_ref[...], preferred_element_type=jnp.float32)
    tn = m * dinv
    out_ref[...] = jnp.concatenate([tn, jnp.zeros_like(tn)], axis=1)


def _final_body(agg_ref, t_ref, dinv_ref, b_ref, out_ref):
    dinv = dinv_ref[:, 0:1]
    u = ((agg_ref[0] + agg_ref[1] + t_ref[...]) * dinv)[:, :64] + b_ref[...]
    u = u - jnp.max(u, axis=1, keepdims=True)
    out_ref[...] = u - jnp.log(jnp.sum(jnp.exp(u), axis=1, keepdims=True))


def _row_spec(w):
    return pl.BlockSpec((BT, w), lambda i: (i, 0))


def _half_spec(w):
    return pl.BlockSpec((NCORE, BT, w), lambda i: (0, i, 0))


def _full_spec(shape):
    return pl.BlockSpec(shape, lambda i: tuple(0 for _ in shape))


def _tc_prep(deg2, xp):
    return pl.pallas_call(
        _prep_body,
        grid=(GT,),
        in_specs=[_half_spec(W), _row_spec(128)],
        out_specs=[_row_spec(16), _row_spec(128)],
        out_shape=[
            jax.ShapeDtypeStruct((NP, 16), jnp.float32),
            jax.ShapeDtypeStruct((NP, 128), jnp.float32),
        ],
    )(deg2, xp)


def _tc_layer1(agg, t, dinv, w, b):
    return pl.pallas_call(
        _layer1_body,
        grid=(GT,),
        in_specs=[
            _half_spec(W),
            _row_spec(128),
            _row_spec(16),
            _full_spec((128, 256)),
            _full_spec((1, 256)),
        ],
        out_specs=_half_spec(W),
        out_shape=jax.ShapeDtypeStruct((NCORE, NP, W), jnp.float32),
    )(agg, t, dinv, w, b)


def _tc_layer(agg, t, dinv, w, b):
    return pl.pallas_call(
        _layer_body,
        grid=(GT,),
        in_specs=[
            _half_spec(W),
            _half_spec(W),
            _row_spec(16),
            _full_spec((256, 256)),
            _full_spec((1, 256)),
        ],
        out_specs=_half_spec(W),
        out_shape=jax.ShapeDtypeStruct((NCORE, NP, W), jnp.float32),
    )(agg, t, dinv, w, b)


def _tc_layer4(agg, t, dinv, w, b, w5):
    return pl.pallas_call(
        _layer4_body,
        grid=(GT,),
        in_specs=[
            _half_spec(W),
            _half_spec(W),
            _row_spec(16),
            _full_spec((256, 256)),
            _full_spec((1, 256)),
            _full_spec((256, 64)),
        ],
        out_specs=_row_spec(128),
        out_shape=jax.ShapeDtypeStruct((NP, 128), jnp.float32),
    )(agg, t, dinv, w, b, w5)


def _tc_final(agg, t, dinv, b):
    return pl.pallas_call(
        _final_body,
        grid=(GT,),
        in_specs=[_half_spec(W), _row_spec(128), _row_spec(16), _full_spec((1, 64))],
        out_specs=_row_spec(64),
        out_shape=jax.ShapeDtypeStruct((NP, 64), jnp.float32),
    )(agg, t, dinv, b)


def kernel(x, edge_index, W1, b1, g1, be1, W2, b2, g2, be2, W3, b3, g3, be3,
           W4, b4, g4, be4, W5, b5):
    src = edge_index[0].astype(jnp.int32)
    dst = edge_index[1].astype(jnp.int32)
    padi = jnp.full((EP - E,), NP - 1, dtype=jnp.int32)
    srcp = jnp.concatenate([src, padi])
    dstp = jnp.concatenate([dst, padi])
    srcs1 = srcp.reshape(ER, 128)
    srcs2 = jnp.stack([srcp, srcp + NP]).reshape(NCORE, ER, 128)
    dsts = dstp.reshape(ER, 128)

    xp = jnp.pad(x, ((0, NP - N), (0, 0)))
    ones = jnp.ones((128, W), jnp.float32)
    zeros = jnp.zeros((128, W), jnp.float32)

    def fold(wi, bi, gi, bei):
        sc = gi / jnp.sqrt(1.0 + BN_EPS)
        return wi * sc[None, :], (bi * sc + bei)[None, :]

    W1f, b1f = fold(W1, b1, g1, be1)
    W2f, b2f = fold(W2, b2, g2, be2)
    W3f, b3f = fold(W3, b3, g3, be3)
    W4f, b4f = fold(W4, b4, g4, be4)

    scat_cols, scat_edges, deg_kernel = _sc_kernels()

    deg2 = deg_kernel(dsts, ones, zeros)
    dinv, t1 = _tc_prep(deg2, xp)

    agg1 = scat_edges(t1, srcs1, dsts, zeros)
    t2 = _tc_layer1(agg1, t1, dinv, W1f, b1f)

    agg2 = scat_cols(t2.reshape(NCORE * NP, W), srcs2, dsts, zeros)
    t3 = _tc_layer(agg2, t2, dinv, W2f, b2f)

    agg3 = scat_cols(t3.reshape(NCORE * NP, W), srcs2, dsts, zeros)
    t4 = _tc_layer(agg3, t3, dinv, W3f, b3f)

    agg4 = scat_cols(t4.reshape(NCORE * NP, W), srcs2, dsts, zeros)
    t5 = _tc_layer4(agg4, t4, dinv, W4f, b4f, W5)

    agg5 = scat_edges(t5, srcs1, dsts, zeros)
    out = _tc_final(agg5, t5, dinv, b5.reshape(1, 64))
    return out[:N]
